# Initial kernel scaffold; baseline (speedup 1.0000x reference)
import jax
import jax.numpy as jnp
from jax.experimental import pallas as pl


def kernel(adj, features, W1, b1, W2, b2, W3, b3, g1, be1, g2, be2, g3, be3, Wa, ba, Wb, bb):
    raise NotImplementedError("write your pallas kernel here")


# speedup vs baseline: 11.9456x; 11.9456x over previous
import functools

import jax
import jax.numpy as jnp
from jax import lax
from jax.experimental import pallas as pl
from jax.experimental.pallas import tpu as pltpu
from jax.experimental.pallas import tpu_sc as plsc

N = 10000
E = 320000
D = 128
H = 64
C = 10

NC = 2
NS = 16
NW = NC * NS
B = 128
E_PAD = ((E + NW * B - 1) // (NW * B)) * (NW * B)
EW = E_PAD // NW
STEPS = EW // B
RPT = (N // NS + 8) // 8 * 8
N_PAD = NS * RPT
DW = 16

_sc_mesh = plsc.VectorSubcoreMesh(core_axis_name="c", subcore_axis_name="s")


@functools.partial(
    pl.kernel,
    out_type=jax.ShapeDtypeStruct((NC, N_PAD, DW), jnp.float32),
    mesh=_sc_mesh,
    scratch_types=[
        pltpu.VMEM((B,), jnp.int32),
        pltpu.VMEM((B, DW), jnp.float32),
        pltpu.VMEM_SHARED((N_PAD, DW), jnp.float32),
    ],
)
def _sc_degree(dst_hbm, ones_hbm, zeros_hbm, out_hbm, dst_v, ones_v, acc_sh):
    c = lax.axis_index("c")
    s = lax.axis_index("s")
    wid = s * NC + c
    pltpu.sync_copy(ones_hbm, ones_v)
    pltpu.sync_copy(zeros_hbm, acc_sh.at[pl.ds(s * RPT, RPT)])
    plsc.subcore_barrier()

    def body(i, carry):
        base = wid * EW + i * B
        pltpu.sync_copy(dst_hbm.at[pl.ds(base, B)], dst_v)
        pltpu.sync_copy(ones_v, acc_sh.at[dst_v], add=True)
        return carry

    lax.fori_loop(0, STEPS, body, 0)
    plsc.subcore_barrier()
    pltpu.sync_copy(acc_sh.at[pl.ds(s * RPT, RPT)],
                    out_hbm.at[c, pl.ds(s * RPT, RPT)])


@functools.partial(
    pl.kernel,
    out_type=jax.ShapeDtypeStruct((NC, N_PAD, H), jnp.float32),
    mesh=_sc_mesh,
    scratch_types=[
        pltpu.VMEM((B,), jnp.int32),
        pltpu.VMEM((B,), jnp.int32),
        pltpu.VMEM((B, H), jnp.float32),
        pltpu.SemaphoreType.DMA,
        pltpu.VMEM_SHARED((N_PAD, H), jnp.float32),
    ],
    compiler_params=pltpu.CompilerParams(use_tc_tiling_on_sc=False),
)
def _sc_scatter(hs_hbm, src_hbm, dst_hbm, zeros_hbm, out_hbm,
                src_v, dst_v, rows_v, sem, acc_sh):
    c = lax.axis_index("c")
    s = lax.axis_index("s")
    wid = s * NC + c
    pltpu.sync_copy(zeros_hbm, acc_sh.at[pl.ds(s * RPT, RPT)])
    plsc.subcore_barrier()

    def body(i, carry):
        base = wid * EW + i * B
        pltpu.sync_copy(src_hbm.at[pl.ds(base, B)], src_v)
        pltpu.sync_copy(dst_hbm.at[pl.ds(base, B)], dst_v)
        pltpu.async_copy(hs_hbm.at[src_v], rows_v, sem).wait()
        pltpu.sync_copy(rows_v, acc_sh.at[dst_v], add=True)
        return carry

    lax.fori_loop(0, STEPS, body, 0)
    plsc.subcore_barrier()
    pltpu.sync_copy(acc_sh.at[pl.ds(s * RPT, RPT)],
                    out_hbm.at[c, pl.ds(s * RPT, RPT)])


def _dinv_body(deg_ref, out_ref):
    d = deg_ref[0, :, 0:1] + deg_ref[1, :, 0:1] + 1.0
    out_ref[...] = lax.rsqrt(d[:N])


def _tc_dinv(degparts):
    return pl.pallas_call(
        _dinv_body,
        out_shape=jax.ShapeDtypeStruct((N, 1), jnp.float32),
    )(degparts)


def _mm_body(x_ref, w_ref, dinv_ref, out_ref):
    hs = jnp.dot(x_ref[...], w_ref[...], precision=lax.Precision.HIGHEST,
                 preferred_element_type=jnp.float32) * dinv_ref[...]
    out_ref[0:N, :] = hs
    out_ref[N:N_PAD, :] = jnp.zeros((N_PAD - N, H), jnp.float32)


def _tc_matmul_scale(x, W, dinv):
    return pl.pallas_call(
        _mm_body,
        out_shape=jax.ShapeDtypeStruct((N_PAD, H), jnp.float32),
    )(x, W, dinv)


def _combine_body(has_res, acc_ref, hs_ref, dinv_ref, b_ref, g_ref, be_ref,
                  *rest):
    if has_res:
        res_ref, out_ref = rest
    else:
        (out_ref,) = rest
    a = acc_ref[0, 0:N, :] + acc_ref[1, 0:N, :] + hs_ref[0:N, :]
    y = dinv_ref[...] * a + b_ref[...]
    mu = y.mean(axis=-1, keepdims=True)
    var = ((y - mu) ** 2).mean(axis=-1, keepdims=True)
    y = (y - mu) * lax.rsqrt(var + 1e-5) * g_ref[...] + be_ref[...]
    y = jnp.maximum(y, 0.0)
    if has_res:
        y = y + res_ref[...]
    out_ref[...] = y


def _tc_combine(acc, hs, dinv, b, g, be, res):
    args = [acc, hs, dinv, b.reshape(1, H), g.reshape(1, H), be.reshape(1, H)]
    if res is not None:
        args.append(res)
    return pl.pallas_call(
        functools.partial(_combine_body, res is not None),
        out_shape=jax.ShapeDtypeStruct((N, H), jnp.float32),
    )(*args)


def _head_body(h_ref, wa_ref, ba_ref, wb_ref, bb_ref, out_ref):
    h = h_ref[...]
    gr = jnp.concatenate([h.mean(axis=0, keepdims=True),
                          h.max(axis=0, keepdims=True)], axis=1)
    mid = jnp.maximum(
        jnp.dot(gr, wa_ref[...], precision=lax.Precision.HIGHEST,
                preferred_element_type=jnp.float32)
        + ba_ref[...], 0.0)
    logits = jnp.dot(mid, wb_ref[...], precision=lax.Precision.HIGHEST,
                     preferred_element_type=jnp.float32) + bb_ref[...]
    m = logits.max(axis=-1, keepdims=True)
    z = logits - m
    lse = jnp.log(jnp.exp(z).sum(axis=-1, keepdims=True))
    out_ref[...] = z - lse


def _tc_head(h, Wa, ba, Wb, bb):
    return pl.pallas_call(
        _head_body,
        out_shape=jax.ShapeDtypeStruct((1, C), jnp.float32),
    )(h, Wa, ba.reshape(1, H), Wb, bb.reshape(1, C))


def kernel(adj, features, W1, b1, W2, b2, W3, b3, g1, be1, g2, be2, g3, be3,
           Wa, ba, Wb, bb):
    src = adj[0].astype(jnp.int32)
    dst = adj[1].astype(jnp.int32)
    pad = E_PAD - E
    srcp = jnp.concatenate([src, jnp.full((pad,), N, jnp.int32)])
    dstp = jnp.concatenate([dst, jnp.full((pad,), N, jnp.int32)])
    ones_deg = jnp.ones((B, DW), jnp.float32)
    zeros_deg = jnp.zeros((RPT, DW), jnp.float32)
    zeros_h = jnp.zeros((RPT, H), jnp.float32)

    degparts = _sc_degree(dstp, ones_deg, zeros_deg)
    dinv = _tc_dinv(degparts)

    h = features
    for (W, b, g, be, has_res) in ((W1, b1, g1, be1, False),
                                   (W2, b2, g2, be2, True),
                                   (W3, b3, g3, be3, True)):
        hs = _tc_matmul_scale(h, W, dinv)
        acc = _sc_scatter(hs, srcp, dstp, zeros_h)
        h = _tc_combine(acc, hs, dinv, b, g, be, h if has_res else None)

    return _tc_head(h, Wa, ba, Wb, bb)
